# baseline (speedup 1.0000x reference)
import functools

import jax
import jax.numpy as jnp
from jax import lax
from jax.experimental import pallas as pl
from jax.experimental.pallas import tpu as pltpu
from jax.experimental.pallas import tpu_sc as plsc

N_CODES = 8192
DIM = 256
TM = 2048
TN = 4096


def _argmin_body(z_ref, w_ref, idx_ref, lsum_ref, w2_ref):
    i = pl.program_id(0)
    zb = z_ref[...]
    zs = zb * (-2.0)
    sumz2 = jnp.sum(zb ** 2, axis=1, keepdims=True)

    @pl.when(i == 0)
    def _():
        for j in range(N_CODES // TN):
            wb = w_ref[j * TN:(j + 1) * TN, :]
            w2_ref[0, j * TN:(j + 1) * TN] = jnp.sum(wb ** 2, axis=1)

    run_min = jnp.full((TM,), jnp.inf, dtype=jnp.float32)
    run_idx = jnp.zeros((TM,), dtype=jnp.float32)
    col = lax.broadcasted_iota(jnp.int32, (1, TN), 1).astype(jnp.float32)
    for j in range(N_CODES // TN):
        wb = w_ref[j * TN:(j + 1) * TN, :]
        mneg2 = lax.dot_general(zs, wb, (((1,), (1,)), ((), ())),
                                preferred_element_type=jnp.float32)
        w2 = w2_ref[0, j * TN:(j + 1) * TN]
        d = (sumz2 + mneg2) + w2[None, :]
        bm = jnp.min(d, axis=1)
        ba = jnp.min(jnp.where(d == bm[:, None], col, jnp.float32(N_CODES)), axis=1)
        upd = bm < run_min
        run_min = jnp.where(upd, bm, run_min)
        run_idx = jnp.where(upd, ba + float(j * TN), run_idx)
    idx_ref[0] = run_idx.astype(jnp.int32).reshape(TM // 128, 128)
    blk_sum = jnp.sum(run_min).reshape(1, 1)

    @pl.when(i == 0)
    def _():
        lsum_ref[...] = blk_sum

    @pl.when(i != 0)
    def _():
        lsum_ref[...] = lsum_ref[...] + blk_sum


def _distance_argmin(z_flat, W):
    n_tok = z_flat.shape[0]
    grid = n_tok // TM
    idx, lsum = pl.pallas_call(
        _argmin_body,
        grid=(grid,),
        in_specs=[
            pl.BlockSpec((TM, DIM), lambda i: (i, 0)),
            pl.BlockSpec((N_CODES, DIM), lambda i: (0, 0)),
        ],
        out_specs=[
            pl.BlockSpec((1, TM // 128, 128), lambda i: (i, 0, 0)),
            pl.BlockSpec((1, 1), lambda i: (0, 0)),
        ],
        out_shape=[
            jax.ShapeDtypeStruct((grid, TM // 128, 128), jnp.int32),
            jax.ShapeDtypeStruct((1, 1), jnp.float32),
        ],
        scratch_shapes=[pltpu.VMEM((1, N_CODES), jnp.float32)],
        compiler_params=pltpu.CompilerParams(
            vmem_limit_bytes=100 * 1024 * 1024),
    )(z_flat, W)
    return idx.reshape(-1), lsum[0, 0]


def _make_sc_gather(n_tok):
    C = 128
    NW = 32
    b_per_w = n_tok // NW
    mesh = plsc.VectorSubcoreMesh(core_axis_name="c", subcore_axis_name="s")

    nch = b_per_w // C

    @functools.partial(
        pl.kernel, mesh=mesh,
        out_type=jax.ShapeDtypeStruct((n_tok, DIM), jnp.float32),
        scratch_types=[
            pltpu.VMEM((b_per_w,), jnp.int32),
            pltpu.VMEM((2, C, DIM), jnp.float32),
            pltpu.SemaphoreType.DMA,
            pltpu.SemaphoreType.DMA,
            pltpu.SemaphoreType.DMA,
            pltpu.SemaphoreType.DMA,
        ],
    )
    def gather_k(table_hbm, idx_hbm, out_hbm, idx_v, rows_v,
                 semg0, semg1, semw0, semw1):
        wid = lax.axis_index("s") * 2 + lax.axis_index("c")
        base = wid * b_per_w
        semg = [semg0, semg1]
        semw = [semw0, semw1]
        pltpu.sync_copy(idx_hbm.at[pl.ds(base, b_per_w)], idx_v)
        g = [None, None]
        wo = [None, None]
        g[0] = pltpu.async_copy(
            table_hbm.at[idx_v.at[pl.ds(0, C)]], rows_v.at[0], semg[0])
        for c in range(nch):
            b = c & 1
            nb = 1 - b
            if c + 1 < nch:
                if wo[nb] is not None:
                    wo[nb].wait()
                g[nb] = pltpu.async_copy(
                    table_hbm.at[idx_v.at[pl.ds((c + 1) * C, C)]],
                    rows_v.at[nb], semg[nb])
            g[b].wait()
            wo[b] = pltpu.async_copy(
                rows_v.at[b], out_hbm.at[pl.ds(base + c * C, C)], semw[b])
        wo[0].wait()
        wo[1].wait()

    return gather_k


def kernel(z, W):
    z_flat = z.reshape(-1, DIM)
    n_tok = z_flat.shape[0]
    idx, lsum = _distance_argmin(z_flat, W)
    z_q = _make_sc_gather(n_tok)(W, idx)
    z_q_st = z_q.reshape(z.shape)
    loss = 1.25 * (lsum / (n_tok * DIM))
    return (z_q_st, loss, idx)


# speedup vs baseline: 1.0048x; 1.0048x over previous
import functools

import jax
import jax.numpy as jnp
from jax import lax
from jax.experimental import pallas as pl
from jax.experimental.pallas import tpu as pltpu
from jax.experimental.pallas import tpu_sc as plsc

N_CODES = 8192
DIM = 256
TM = 2048
TN = 2048


def _argmin_body(z_ref, w_ref, idx_ref, lsum_ref, w2_ref):
    i = pl.program_id(0)
    zb = z_ref[...]
    zs = zb * (-2.0)
    sumz2 = jnp.sum(zb ** 2, axis=1, keepdims=True)

    @pl.when(i == 0)
    def _():
        for j in range(N_CODES // TN):
            wb = w_ref[j * TN:(j + 1) * TN, :]
            w2_ref[0, j * TN:(j + 1) * TN] = jnp.sum(wb ** 2, axis=1)

    run_min = jnp.full((TM,), jnp.inf, dtype=jnp.float32)
    run_idx = jnp.zeros((TM,), dtype=jnp.float32)
    col = lax.broadcasted_iota(jnp.int32, (1, TN), 1).astype(jnp.float32)
    for j in range(N_CODES // TN):
        wb = w_ref[j * TN:(j + 1) * TN, :]
        mneg2 = lax.dot_general(zs, wb, (((1,), (1,)), ((), ())),
                                preferred_element_type=jnp.float32)
        w2 = w2_ref[0, j * TN:(j + 1) * TN]
        d = (sumz2 + mneg2) + w2[None, :]
        bm = jnp.min(d, axis=1)
        ba = jnp.min(jnp.where(d == bm[:, None], col, jnp.float32(N_CODES)), axis=1)
        upd = bm < run_min
        run_min = jnp.where(upd, bm, run_min)
        run_idx = jnp.where(upd, ba + float(j * TN), run_idx)
    idx_ref[0] = run_idx.astype(jnp.int32).reshape(TM // 128, 128)
    blk_sum = jnp.sum(run_min).reshape(1, 1)

    @pl.when(i == 0)
    def _():
        lsum_ref[...] = blk_sum

    @pl.when(i != 0)
    def _():
        lsum_ref[...] = lsum_ref[...] + blk_sum


def _distance_argmin(z_flat, W):
    n_tok = z_flat.shape[0]
    grid = n_tok // TM
    idx, lsum = pl.pallas_call(
        _argmin_body,
        grid=(grid,),
        in_specs=[
            pl.BlockSpec((TM, DIM), lambda i: (i, 0)),
            pl.BlockSpec((N_CODES, DIM), lambda i: (0, 0)),
        ],
        out_specs=[
            pl.BlockSpec((1, TM // 128, 128), lambda i: (i, 0, 0)),
            pl.BlockSpec((1, 1), lambda i: (0, 0)),
        ],
        out_shape=[
            jax.ShapeDtypeStruct((grid, TM // 128, 128), jnp.int32),
            jax.ShapeDtypeStruct((1, 1), jnp.float32),
        ],
        scratch_shapes=[pltpu.VMEM((1, N_CODES), jnp.float32)],
        compiler_params=pltpu.CompilerParams(
            vmem_limit_bytes=100 * 1024 * 1024),
    )(z_flat, W)
    return idx.reshape(-1), lsum[0, 0]


def _make_sc_gather(n_tok):
    C = 128
    NW = 32
    b_per_w = n_tok // NW
    mesh = plsc.VectorSubcoreMesh(core_axis_name="c", subcore_axis_name="s")

    nch = b_per_w // C

    @functools.partial(
        pl.kernel, mesh=mesh,
        out_type=jax.ShapeDtypeStruct((n_tok, DIM), jnp.float32),
        scratch_types=[
            pltpu.VMEM((b_per_w,), jnp.int32),
            pltpu.VMEM((2, C, DIM), jnp.float32),
            pltpu.SemaphoreType.DMA,
            pltpu.SemaphoreType.DMA,
            pltpu.SemaphoreType.DMA,
            pltpu.SemaphoreType.DMA,
        ],
    )
    def gather_k(table_hbm, idx_hbm, out_hbm, idx_v, rows_v,
                 semg0, semg1, semw0, semw1):
        wid = lax.axis_index("s") * 2 + lax.axis_index("c")
        base = wid * b_per_w
        semg = [semg0, semg1]
        semw = [semw0, semw1]
        pltpu.sync_copy(idx_hbm.at[pl.ds(base, b_per_w)], idx_v)
        g = [None, None]
        wo = [None, None]
        g[0] = pltpu.async_copy(
            table_hbm.at[idx_v.at[pl.ds(0, C)]], rows_v.at[0], semg[0])
        for c in range(nch):
            b = c & 1
            nb = 1 - b
            if c + 1 < nch:
                if wo[nb] is not None:
                    wo[nb].wait()
                g[nb] = pltpu.async_copy(
                    table_hbm.at[idx_v.at[pl.ds((c + 1) * C, C)]],
                    rows_v.at[nb], semg[nb])
            g[b].wait()
            wo[b] = pltpu.async_copy(
                rows_v.at[b], out_hbm.at[pl.ds(base + c * C, C)], semw[b])
        wo[0].wait()
        wo[1].wait()

    return gather_k


def kernel(z, W):
    z_flat = z.reshape(-1, DIM)
    n_tok = z_flat.shape[0]
    idx, lsum = _distance_argmin(z_flat, W)
    z_q = _make_sc_gather(n_tok)(W, idx)
    z_q_st = z_q.reshape(z.shape)
    loss = 1.25 * (lsum / (n_tok * DIM))
    return (z_q_st, loss, idx)
